# baseline (speedup 1.0000x reference)
import jax
import jax.numpy as jnp
from jax import lax
from jax.experimental import pallas as pl
from jax.experimental.pallas import tpu as pltpu
from jax.experimental.pallas import tpu_sc as plsc

_NUM_CORES = 2
_NUM_SUBCORES = 16
_NUM_WORKERS = _NUM_CORES * _NUM_SUBCORES
_DEPTH = 10


def kernel(x, weight):
    b, h = x.shape
    n = b * h
    dim = weight.shape[1]
    cols = b // _NUM_WORKERS
    half = cols // 2
    n_chunks = 2 * h
    idx2 = x.T

    mesh = plsc.VectorSubcoreMesh(core_axis_name="c", subcore_axis_name="s")

    @pl.kernel(
        out_type=jax.ShapeDtypeStruct((n, dim), weight.dtype),
        mesh=mesh,
        scratch_types=[pltpu.VMEM((h, cols), jnp.int32)]
        + [pltpu.VMEM((half, dim), jnp.float32)] * _DEPTH
        + [pltpu.SemaphoreType.DMA] * (2 * _DEPTH),
    )
    def gather_kernel(w_hbm, i_hbm, o_hbm, idx_v, *bufs_sems):
        bufs = bufs_sems[:_DEPTH]
        semg = bufs_sems[_DEPTH : 2 * _DEPTH]
        semw = bufs_sems[2 * _DEPTH :]
        wid = lax.axis_index("s") * _NUM_CORES + lax.axis_index("c")
        col0 = wid * cols
        pltpu.sync_copy(i_hbm.at[:, pl.ds(col0, cols)], idx_v)

        def gather_start(c, k):
            r = c // 2
            s = (c % 2) * half
            pltpu.async_copy(
                w_hbm.at[idx_v.at[r, pl.ds(s, half)]], bufs[k], semg[k]
            )

        def gather_wait(c, k):
            r = c // 2
            s = (c % 2) * half
            pltpu.make_async_copy(
                w_hbm.at[idx_v.at[r, pl.ds(s, half)]], bufs[k], semg[k]
            ).wait()

        def wb_start(c, k):
            r = c // 2
            s = (c % 2) * half
            pltpu.async_copy(
                bufs[k], o_hbm.at[pl.ds(r * b + col0 + s, half)], semw[k]
            )

        def wb_drain(c, k):
            r = c // 2
            s = (c % 2) * half
            pltpu.make_async_copy(
                bufs[k], o_hbm.at[pl.ds(r * b + col0 + s, half)], semw[k]
            ).wait()

        for k in range(_DEPTH):
            gather_start(k, k)

        @pl.loop(0, n_chunks, step=_DEPTH)
        def _(c):
            for k in range(_DEPTH):
                gather_wait(c + k, k)
                wb_start(c + k, k)
                wb_drain(c + k, k)

                @pl.when(c + k + _DEPTH < n_chunks)
                def _(c=c, k=k):
                    gather_start(c + k + _DEPTH, k)

    out = gather_kernel(weight, idx2)
    return out.reshape(h, b, dim).transpose(1, 0, 2)
